# baseline (speedup 1.0000x reference)
import functools

import jax
import jax.numpy as jnp
from jax import lax
from jax.experimental import pallas as pl
from jax.experimental.pallas import tpu as pltpu
from jax.experimental.pallas import tpu_sc as plsc

LAMBDA = 1.0

_NC = 1
_NS = 16
_NW = _NC * _NS
_L = 16

_ROWS = 16384
_D = 128
_SC_ROWS = 6144
_TC_ROWS = _ROWS - _SC_ROWS
_RPW = _SC_ROWS // _NW
_CHUNKS = _D // _L
_SCALE = LAMBDA * 0.5 / _ROWS

_TC_BLOCK = 2048
assert _SC_ROWS % _TC_BLOCK == 0 and _TC_ROWS % _TC_BLOCK == 0
_TC_OFF = _SC_ROWS // _TC_BLOCK
_SLABS = _TC_BLOCK // _D


def _make_sc_partials():
    mesh = plsc.VectorSubcoreMesh(core_axis_name="c", subcore_axis_name="s",
                                  num_cores=_NC, num_subcores=_NS)

    @functools.partial(
        pl.kernel,
        mesh=mesh,
        out_type=jax.ShapeDtypeStruct((_NW, _L), jnp.float32),
        scratch_types=[
            pltpu.VMEM((_RPW, _D), jnp.float32),
            pltpu.VMEM((_RPW,), jnp.int32),
            pltpu.VMEM((_RPW, _L), jnp.float32),
            pltpu.VMEM((1, _D), jnp.float32),
            pltpu.VMEM((1, _D), jnp.float32),
            pltpu.VMEM((_L,), jnp.float32),
            pltpu.SemaphoreType.DMA,
            pltpu.SemaphoreType.DMA,
        ],
    )
    def sc_partials(feat_hbm, lab_hbm, c0_hbm, c1_hbm, out_hbm,
                    feat_v, lab_v, lab16_v, c0_v, c1_v, acc_v,
                    sem_a, sem_b):
        wid = lax.axis_index("s") * _NC + lax.axis_index("c")
        base = wid * _RPW
        hf = pltpu.async_copy(feat_hbm.at[pl.ds(base, _RPW)], feat_v,
                              sem_a)
        hl = pltpu.async_copy(lab_hbm.at[pl.ds(base, _RPW)], lab_v,
                              sem_b)
        h0 = pltpu.async_copy(c0_hbm, c0_v, sem_b)
        h1 = pltpu.async_copy(c1_hbm, c1_v, sem_b)
        hl.wait()
        h0.wait()
        h1.wait()

        c0 = [c0_v[0, pl.ds(j * _L, _L)] for j in range(_CHUNKS)]
        dlt = [c1_v[0, pl.ds(j * _L, _L)] - c0[j] for j in range(_CHUNKS)]

        def expand_body(g, _):
            lvf = lab_v[pl.ds(g * _L, _L)].astype(jnp.float32)
            for k in range(_L):
                lab16_v[g * _L + k, :] = jnp.full((_L,), lvf[k],
                                                  jnp.float32)
            return 0

        lax.fori_loop(0, _RPW // _L, expand_body, 0)
        hf.wait()

        def row_body(r, accs):
            lf = lab16_v[r, :]
            a0, a1 = accs
            for j in range(_CHUNKS):
                t = feat_v[r, pl.ds(j * _L, _L)] - c0[j] - lf * dlt[j]
                if j % 2 == 0:
                    a0 = a0 + t * t
                else:
                    a1 = a1 + t * t
            return a0, a1

        zero = jnp.zeros((_L,), jnp.float32)
        a0, a1 = lax.fori_loop(0, _RPW, row_body, (zero, zero),
                               unroll=4)
        acc_v[...] = (a0 + a1) * _SCALE
        pltpu.sync_copy(acc_v, out_hbm.at[wid])

    return sc_partials


_sc_partials = _make_sc_partials()


def _tc_body(feat_ref, lab_ref, q_ref, n1_ref, s_ref, t_ref):
    i = pl.program_id(0)
    f = feat_ref[...]
    lab = lab_ref[...].astype(jnp.float32)

    @pl.when(i == 0)
    def _():
        q_ref[0, 0] = 0.0
        n1_ref[0, 0] = 0.0
        s_ref[...] = jnp.zeros_like(s_ref)
        t_ref[...] = jnp.zeros_like(t_ref)

    q_ref[0, 0] += jnp.sum(f * f)
    n1_ref[0, 0] += jnp.sum(lab)
    s_ref[...] += jnp.sum(f, axis=0, keepdims=True)
    f3 = f.reshape(_SLABS, _D, _D)
    t = jnp.zeros((1, _D), jnp.float32)
    for s in range(_SLABS):
        t = t + jax.lax.dot(lab[s:s + 1, :], f3[s],
                            preferred_element_type=jnp.float32)
    t_ref[...] += t


def _tc_moments(feat, labf, ):
    nb = _TC_ROWS // _TC_BLOCK
    return pl.pallas_call(
        _tc_body,
        grid=(nb,),
        in_specs=[
            pl.BlockSpec((_TC_BLOCK, _D), lambda i: (i + _TC_OFF, 0)),
            pl.BlockSpec((_SLABS, _D), lambda i: (i + _TC_OFF, 0)),
        ],
        out_specs=[
            pl.BlockSpec(block_shape=(1, 1), index_map=lambda i: (0, 0),
                         memory_space=pltpu.SMEM),
            pl.BlockSpec(block_shape=(1, 1), index_map=lambda i: (0, 0),
                         memory_space=pltpu.SMEM),
            pl.BlockSpec(block_shape=(1, _D), index_map=lambda i: (0, 0)),
            pl.BlockSpec(block_shape=(1, _D), index_map=lambda i: (0, 0)),
        ],
        out_shape=[
            jax.ShapeDtypeStruct((1, 1), jnp.float32),
            jax.ShapeDtypeStruct((1, 1), jnp.float32),
            jax.ShapeDtypeStruct((1, _D), jnp.float32),
            jax.ShapeDtypeStruct((1, _D), jnp.float32),
        ],
    )(feat, labf)


def kernel(features, labels, proto_0, proto_1):
    labels = labels.astype(jnp.int32)
    sc_part = _sc_partials(features, labels, proto_0, proto_1)
    q, n1, s, t = _tc_moments(features, labels.reshape(_ROWS // _D, _D))
    n_tc = jnp.float32(_TC_ROWS)
    cross = jnp.sum(s * proto_0) + jnp.sum(t * (proto_1 - proto_0))
    norms = ((n_tc - n1[0, 0]) * jnp.sum(proto_0 * proto_0)
             + n1[0, 0] * jnp.sum(proto_1 * proto_1))
    tc_loss = _SCALE * (q[0, 0] - 2.0 * cross + norms)
    return jnp.sum(sc_part) + tc_loss


# speedup vs baseline: 1.0056x; 1.0056x over previous
import functools

import jax
import jax.numpy as jnp
from jax import lax
from jax.experimental import pallas as pl
from jax.experimental.pallas import tpu as pltpu
from jax.experimental.pallas import tpu_sc as plsc

LAMBDA = 1.0

_NC = 1
_NS = 16
_NW = _NC * _NS
_L = 16

_ROWS = 16384
_D = 128
_SC_ROWS = 6144
_TC_ROWS = _ROWS - _SC_ROWS
_RPW = _SC_ROWS // _NW
_CHUNKS = _D // _L
_SCALE = LAMBDA * 0.5 / _ROWS

_TC_BLOCK = 2048
assert _SC_ROWS % _TC_BLOCK == 0 and _TC_ROWS % _TC_BLOCK == 0
_TC_OFF = _SC_ROWS // _TC_BLOCK
_SLABS = _TC_BLOCK // _D


def _make_sc_partials():
    mesh = plsc.VectorSubcoreMesh(core_axis_name="c", subcore_axis_name="s",
                                  num_cores=_NC, num_subcores=_NS)

    @functools.partial(
        pl.kernel,
        mesh=mesh,
        out_type=jax.ShapeDtypeStruct((_NW, _L), jnp.float32),
        scratch_types=[
            pltpu.VMEM((_RPW // 2, _D), jnp.float32),
            pltpu.VMEM((_RPW // 2, _D), jnp.float32),
            pltpu.VMEM((_RPW,), jnp.int32),
            pltpu.VMEM((_RPW, _L), jnp.float32),
            pltpu.VMEM((1, _D), jnp.float32),
            pltpu.VMEM((1, _D), jnp.float32),
            pltpu.VMEM((_L,), jnp.float32),
            pltpu.SemaphoreType.DMA,
            pltpu.SemaphoreType.DMA,
            pltpu.SemaphoreType.DMA,
        ],
    )
    def sc_partials(feat_hbm, lab_hbm, c0_hbm, c1_hbm, out_hbm,
                    feat0_v, feat1_v, lab_v, lab16_v, c0_v, c1_v, acc_v,
                    sem_a, sem_b, sem_c):
        wid = lax.axis_index("s") * _NC + lax.axis_index("c")
        base = wid * _RPW
        half = _RPW // 2
        hfa = pltpu.async_copy(feat_hbm.at[pl.ds(base, half)], feat0_v,
                               sem_a)
        hfb = pltpu.async_copy(feat_hbm.at[pl.ds(base + half, half)],
                               feat1_v, sem_b)
        hl = pltpu.async_copy(lab_hbm.at[pl.ds(base, _RPW)], lab_v,
                              sem_c)
        h0 = pltpu.async_copy(c0_hbm, c0_v, sem_c)
        h1 = pltpu.async_copy(c1_hbm, c1_v, sem_c)
        hl.wait()
        h0.wait()
        h1.wait()

        c0 = [c0_v[0, pl.ds(j * _L, _L)] for j in range(_CHUNKS)]
        dlt = [c1_v[0, pl.ds(j * _L, _L)] - c0[j] for j in range(_CHUNKS)]

        def expand_body(g, _):
            lvf = lab_v[pl.ds(g * _L, _L)].astype(jnp.float32)
            for k in range(_L):
                lab16_v[g * _L + k, :] = jnp.full((_L,), lvf[k],
                                                  jnp.float32)
            return 0

        lax.fori_loop(0, _RPW // _L, expand_body, 0)

        def make_row_body(feat_v, lab_off):
            def row_body(r, accs):
                lf = lab16_v[lab_off + r, :]
                a0, a1 = accs
                for j in range(_CHUNKS):
                    t = (feat_v[r, pl.ds(j * _L, _L)]
                         - c0[j] - lf * dlt[j])
                    if j % 2 == 0:
                        a0 = a0 + t * t
                    else:
                        a1 = a1 + t * t
                return a0, a1
            return row_body

        zero = jnp.zeros((_L,), jnp.float32)
        hfa.wait()
        accs = lax.fori_loop(0, half, make_row_body(feat0_v, 0),
                             (zero, zero), unroll=4)
        hfb.wait()
        a0, a1 = lax.fori_loop(0, half, make_row_body(feat1_v, half),
                               accs, unroll=4)
        acc_v[...] = (a0 + a1) * _SCALE
        pltpu.sync_copy(acc_v, out_hbm.at[wid])

    return sc_partials


_sc_partials = _make_sc_partials()


def _tc_body(feat_ref, lab_ref, q_ref, n1_ref, s_ref, t_ref):
    i = pl.program_id(0)
    f = feat_ref[...]
    lab = lab_ref[...].astype(jnp.float32)

    @pl.when(i == 0)
    def _():
        q_ref[0, 0] = 0.0
        n1_ref[0, 0] = 0.0
        s_ref[...] = jnp.zeros_like(s_ref)
        t_ref[...] = jnp.zeros_like(t_ref)

    q_ref[0, 0] += jnp.sum(f * f)
    n1_ref[0, 0] += jnp.sum(lab)
    s_ref[...] += jnp.sum(f, axis=0, keepdims=True)
    f3 = f.reshape(_SLABS, _D, _D)
    t = jnp.zeros((1, _D), jnp.float32)
    for s in range(_SLABS):
        t = t + jax.lax.dot(lab[s:s + 1, :], f3[s],
                            preferred_element_type=jnp.float32)
    t_ref[...] += t


def _tc_moments(feat, labf, ):
    nb = _TC_ROWS // _TC_BLOCK
    return pl.pallas_call(
        _tc_body,
        grid=(nb,),
        in_specs=[
            pl.BlockSpec((_TC_BLOCK, _D), lambda i: (i + _TC_OFF, 0)),
            pl.BlockSpec((_SLABS, _D), lambda i: (i + _TC_OFF, 0)),
        ],
        out_specs=[
            pl.BlockSpec(block_shape=(1, 1), index_map=lambda i: (0, 0),
                         memory_space=pltpu.SMEM),
            pl.BlockSpec(block_shape=(1, 1), index_map=lambda i: (0, 0),
                         memory_space=pltpu.SMEM),
            pl.BlockSpec(block_shape=(1, _D), index_map=lambda i: (0, 0)),
            pl.BlockSpec(block_shape=(1, _D), index_map=lambda i: (0, 0)),
        ],
        out_shape=[
            jax.ShapeDtypeStruct((1, 1), jnp.float32),
            jax.ShapeDtypeStruct((1, 1), jnp.float32),
            jax.ShapeDtypeStruct((1, _D), jnp.float32),
            jax.ShapeDtypeStruct((1, _D), jnp.float32),
        ],
    )(feat, labf)


def kernel(features, labels, proto_0, proto_1):
    labels = labels.astype(jnp.int32)
    sc_part = _sc_partials(features, labels, proto_0, proto_1)
    q, n1, s, t = _tc_moments(features, labels.reshape(_ROWS // _D, _D))
    n_tc = jnp.float32(_TC_ROWS)
    cross = jnp.sum(s * proto_0) + jnp.sum(t * (proto_1 - proto_0))
    norms = ((n_tc - n1[0, 0]) * jnp.sum(proto_0 * proto_0)
             + n1[0, 0] * jnp.sum(proto_1 * proto_1))
    tc_loss = _SCALE * (q[0, 0] - 2.0 * cross + norms)
    return jnp.sum(sc_part) + tc_loss
